# speedup vs baseline: 8.2995x; 8.2995x over previous
import functools

import jax
import jax.numpy as jnp
from jax import lax
from jax.experimental import pallas as pl
from jax.experimental.pallas import tpu as pltpu
from jax.experimental.pallas import tpu_sc as plsc

V = 100000
D = 256
B = 4096 * 26
NW = 32
C = 128
NCHUNK = B // (NW * C)
BPW = NCHUNK * C


def _sc_gather(table, idx3):
    mesh = plsc.VectorSubcoreMesh(core_axis_name="c", subcore_axis_name="s")

    @functools.partial(
        pl.kernel,
        mesh=mesh,
        out_type=jax.ShapeDtypeStruct((B, D), jnp.float32),
        scratch_types=[
            pltpu.VMEM((NCHUNK, C), jnp.int32),
            pltpu.VMEM((C, D), jnp.float32),
            pltpu.SemaphoreType.DMA,
        ],
    )
    def k(table_hbm, idx_hbm, out_hbm, idx_v, rows_v, sem):
        wid = lax.axis_index("s") * 2 + lax.axis_index("c")
        base = wid * BPW
        pltpu.sync_copy(idx_hbm.at[wid], idx_v)

        def body(j, _):
            pltpu.async_copy(table_hbm.at[idx_v.at[j]], rows_v, sem).wait()
            pltpu.sync_copy(rows_v, out_hbm.at[pl.ds(base + j * C, C)])
            return ()

        lax.fori_loop(0, NCHUNK, body, ())

    return k(table, idx3)


def kernel(inputs, embedding):
    idx3 = inputs.reshape(NW, NCHUNK, C).astype(jnp.int32)
    table = embedding.reshape(V, D)
    out = _sc_gather(table, idx3)
    return out.reshape(4096, 26, 16, 16)


# speedup vs baseline: 8.3896x; 1.0109x over previous
import functools

import jax
import jax.numpy as jnp
from jax import lax
from jax.experimental import pallas as pl
from jax.experimental.pallas import tpu as pltpu
from jax.experimental.pallas import tpu_sc as plsc

V = 100000
D = 256
B = 4096 * 26
NW = 32
C = 104
NCHUNK = B // (NW * C)
BPW = NCHUNK * C
NBUF = 4


def _sc_gather(table, idx3):
    mesh = plsc.VectorSubcoreMesh(core_axis_name="c", subcore_axis_name="s")

    @functools.partial(
        pl.kernel,
        mesh=mesh,
        out_type=jax.ShapeDtypeStruct((B, D), jnp.float32),
        scratch_types=[
            pltpu.VMEM((NCHUNK, C), jnp.int32),
            *[pltpu.VMEM((C, D), jnp.float32) for _ in range(NBUF)],
            *[pltpu.SemaphoreType.DMA for _ in range(2 * NBUF)],
        ],
    )
    def k(table_hbm, idx_hbm, out_hbm, idx_v, *bufs_and_sems):
        rows = bufs_and_sems[:NBUF]
        gsem = bufs_and_sems[NBUF:2 * NBUF]
        wsem = bufs_and_sems[2 * NBUF:]
        wid = lax.axis_index("s") * 2 + lax.axis_index("c")
        base = wid * BPW
        pltpu.sync_copy(idx_hbm.at[wid], idx_v)

        def gather(b, j):
            return pltpu.make_async_copy(
                table_hbm.at[idx_v.at[j]], rows[b], gsem[b])

        def write(b, j):
            return pltpu.make_async_copy(
                rows[b], out_hbm.at[pl.ds(base + j * C, C)], wsem[b])

        for b in range(NBUF):
            gather(b, b).start()

        def outer(s, _):
            for b in range(NBUF):
                j = s * NBUF + b
                gather(b, j).wait()
                write(b, j).start()
            for b in range(NBUF):
                j = s * NBUF + b

                @pl.when(j + NBUF < NCHUNK)
                def _():
                    write(b, j).wait()
                    gather(b, j + NBUF).start()

            return ()

        lax.fori_loop(0, NCHUNK // NBUF, outer, ())

        for b in range(NBUF):
            write(b, NCHUNK - NBUF + b).wait()

    return k(table, idx3)


def kernel(inputs, embedding):
    idx3 = inputs.reshape(NW, NCHUNK, C).astype(jnp.int32)
    table = embedding.reshape(V, D)
    out = _sc_gather(table, idx3)
    return out.reshape(4096, 26, 16, 16)


# speedup vs baseline: 35.9263x; 4.2823x over previous
import functools

import jax
import jax.numpy as jnp
from jax import lax
from jax.experimental import pallas as pl
from jax.experimental.pallas import tpu as pltpu
from jax.experimental.pallas import tpu_sc as plsc

V = 100000
D = 256
NB = 4096
NL = 26
PPC = D // 2
BLK = 8
NBLK = PPC // BLK
LSPLIT = NL // 2


def _sc_gather_t(tableT, idxT):
    mesh = plsc.VectorSubcoreMesh(core_axis_name="c", subcore_axis_name="s")

    @functools.partial(
        pl.kernel,
        mesh=mesh,
        out_type=jax.ShapeDtypeStruct((NL * D, NB), jnp.float32),
        scratch_types=[
            pltpu.VMEM_SHARED((32, NB), jnp.int32),
            pltpu.VMEM((NB,), jnp.int32),
            pltpu.VMEM((NB,), jnp.float32),
            pltpu.VMEM((V,), jnp.float32),
        ],
        compiler_params=pltpu.CompilerParams(needs_layout_passes=False),
    )
    def k(tableT_hbm, idxT_hbm, out_hbm, sh_idx, idx_v, row_v, plane_v):
        cid = lax.axis_index("c")
        sid = lax.axis_index("s")
        wid = cid * 16 + sid

        @pl.when(sid == 0)
        def _():
            pltpu.sync_copy(idxT_hbm, sh_idx)

        plsc.subcore_barrier()

        def plane(j, _):
            p = wid * BLK + j
            pltpu.sync_copy(tableT_hbm.at[p], plane_v)

            plane_lo = plane_v.at[pl.ds(0, 65536)]
            plane_hi = plane_v.at[pl.ds(65536, V - 65536)]

            def do_l(l, _):
                pltpu.sync_copy(sh_idx.at[l], idx_v)

                @plsc.parallel_loop(0, NB // 16, 1, unroll=8)
                def _(kk):
                    bvec = idx_v[pl.ds(kk * 16, 16)]
                    lo = plsc.load_gather(plane_lo, [bvec & 65535])
                    hivec = jnp.clip(bvec - 65536, 0, V - 65536 - 1)
                    hi = plsc.load_gather(plane_hi, [hivec])
                    row_v[pl.ds(kk * 16, 16)] = jnp.where(
                        bvec < 65536, lo, hi)

                pltpu.sync_copy(row_v, out_hbm.at[l * D + p])
                return ()

            lax.fori_loop(0, NL, do_l, ())
            return ()

        lax.fori_loop(0, BLK, plane, ())

    return k(tableT, idxT)


def kernel(inputs, embedding):
    tableT = embedding.reshape(V, D).T
    idxT = jnp.pad(inputs.T.astype(jnp.int32), ((0, 32 - NL), (0, 0)))
    out2 = _sc_gather_t(tableT, idxT)
    outT = out2.reshape(NL, 16, 16, NB)
    return outT.transpose(3, 0, 1, 2)


# speedup vs baseline: 50.9385x; 1.4179x over previous
import functools

import jax
import jax.numpy as jnp
from jax import lax
from jax.experimental import pallas as pl
from jax.experimental.pallas import tpu as pltpu
from jax.experimental.pallas import tpu_sc as plsc

V = 100000
D = 256
NB = 4096
NL = 26
PPC = D // 2
BLK = 8
NBLK = PPC // BLK
LSPLIT = NL // 2


def _sc_gather_t(tableT, idxT):
    mesh = plsc.VectorSubcoreMesh(core_axis_name="c", subcore_axis_name="s")

    @functools.partial(
        pl.kernel,
        mesh=mesh,
        out_type=jax.ShapeDtypeStruct((NL * D, NB), jnp.float32),
        scratch_types=[
            pltpu.VMEM_SHARED((32, NB), jnp.int32),
            *[pltpu.VMEM((NB,), jnp.int32) for _ in range(2)],
            *[pltpu.VMEM((NB,), jnp.float32) for _ in range(2)],
            pltpu.VMEM((V,), jnp.float32),
            *[pltpu.SemaphoreType.DMA for _ in range(4)],
        ],
        compiler_params=pltpu.CompilerParams(needs_layout_passes=False),
    )
    def k(tableT_hbm, idxT_hbm, out_hbm, sh_idx,
          idx0, idx1, row0, row1, plane_v, isem0, isem1, wsem0, wsem1):
        idx_v = (idx0, idx1)
        row_v = (row0, row1)
        isem = (isem0, isem1)
        wsem = (wsem0, wsem1)
        cid = lax.axis_index("c")
        sid = lax.axis_index("s")
        wid = cid * 16 + sid

        @pl.when(sid == 0)
        def _():
            pltpu.sync_copy(idxT_hbm, sh_idx)

        plsc.subcore_barrier()

        def idx_cp(l, par):
            return pltpu.make_async_copy(sh_idx.at[l], idx_v[par], isem[par])

        def wr_cp(l, p, par):
            return pltpu.make_async_copy(
                row_v[par], out_hbm.at[l * D + p], wsem[par])

        def plane(j, _):
            p = wid * BLK + j
            pltpu.sync_copy(tableT_hbm.at[p], plane_v)

            plane_lo = plane_v.at[pl.ds(0, 65536)]
            plane_hi = plane_v.at[pl.ds(65536, V - 65536)]

            idx_cp(0, 0).start()

            def do_l(l, par):
                idx_cp(l, par).wait()

                @pl.when(l + 1 < NL)
                def _():
                    idx_cp(l + 1, 1 - par).start()

                @pl.when(l >= 2)
                def _():
                    wr_cp(l - 2, p, par).wait()

                @plsc.parallel_loop(0, NB // 16, 1, unroll=8)
                def _(kk):
                    bvec = idx_v[par][pl.ds(kk * 16, 16)]
                    m = bvec & 65535
                    lo = plsc.load_gather(plane_lo, [m])
                    hi = plsc.load_gather(
                        plane_hi, [jnp.minimum(m, V - 65536 - 1)])
                    row_v[par][pl.ds(kk * 16, 16)] = jnp.where(
                        bvec < 65536, lo, hi)

                wr_cp(l, p, par).start()

            def pair(ll, _):
                do_l(2 * ll, 0)
                do_l(2 * ll + 1, 1)
                return ()

            lax.fori_loop(0, NL // 2, pair, ())
            wr_cp(NL - 2, p, 0).wait()
            wr_cp(NL - 1, p, 1).wait()
            return ()

        lax.fori_loop(0, BLK, plane, ())

    return k(tableT, idxT)


def kernel(inputs, embedding):
    tableT = embedding.reshape(V, D).T
    idxT = jnp.pad(inputs.T.astype(jnp.int32), ((0, 32 - NL), (0, 0)))
    out2 = _sc_gather_t(tableT, idxT)
    outT = out2.reshape(NL, 16, 16, NB)
    return outT.transpose(3, 0, 1, 2)
